# baseline (speedup 1.0000x reference)
import functools

import jax
import jax.numpy as jnp
from jax import lax
from jax.experimental import pallas as pl
from jax.experimental.pallas import tpu as pltpu
from jax.experimental.pallas import tpu_sc as plsc

NUM_CLASSES = 100
LAMBDA_ = 3.0
BETA = 0.7
LANES = 16
CPAD = 112
NSLOT = 8


def _make_sc_gather(batch, ncls):
    info = plsc.get_sparse_core_info()
    nc, ns = info.num_cores, info.num_subcores
    nw = nc * ns
    assert batch % (LANES * nw) == 0 and LANES % NSLOT == 0
    b_per_w = batch // nw
    nchunk_c = CPAD // LANES
    nchunk_i = b_per_w // LANES
    mesh = plsc.VectorSubcoreMesh(core_axis_name="c", subcore_axis_name="s")

    @functools.partial(
        pl.kernel,
        mesh=mesh,
        out_type=jax.ShapeDtypeStruct((CPAD, batch), jnp.float32),
        scratch_types=[
            pltpu.VMEM((b_per_w,), jnp.int32),
        ]
        + [pltpu.VMEM((ncls, 128), jnp.float32)] * NSLOT
        + [
            pltpu.VMEM((CPAD, b_per_w), jnp.float32),
        ]
        + [pltpu.SemaphoreType.DMA] * NSLOT,
        compiler_params=pltpu.CompilerParams(needs_layout_passes=False),
    )
    def gather_kernel(idx_hbm, table_hbm, out_hbm, idx_v, *rest):
        tiles = rest[:NSLOT]
        rows_t = rest[NSLOT]
        sems = rest[NSLOT + 1 :]
        wid = lax.axis_index("s") * nc + lax.axis_index("c")
        base = wid * b_per_w
        pltpu.sync_copy(idx_hbm.at[pl.ds(base, b_per_w)], idx_v)
        lane = lax.iota(jnp.int32, LANES)

        def col0_of(v):
            return pl.multiple_of((v // 128) * 128, 128)

        def issue(v, k):
            pltpu.async_copy(
                table_hbm.at[:, pl.ds(col0_of(v), 128)],
                tiles[k % NSLOT],
                sems[k % NSLOT],
            )

        def wait(k):
            pltpu.make_async_copy(
                table_hbm.at[:, pl.ds(0, 128)],
                tiles[k % NSLOT],
                sems[k % NSLOT],
            ).wait()

        def extract(v, k, i_local):
            col = jnp.full((LANES,), v - col0_of(v), jnp.int32)
            out_col = jnp.full((LANES,), i_local, jnp.int32)
            for j in range(nchunk_c):
                row = jnp.minimum(lane + (j * LANES), ncls - 1)
                g = plsc.load_gather(tiles[k % NSLOT], [row, col])
                plsc.store_scatter(rows_t, [lane + (j * LANES), out_col], g)

        vec0 = idx_v[pl.ds(0, LANES)]
        for k in range(NSLOT - 1):
            issue(vec0[k], k)

        def body(g, carry):
            vec = idx_v[pl.ds(g * LANES, LANES)]
            vec_next = idx_v[
                pl.ds(jnp.minimum(g + 1, nchunk_i - 1) * LANES, LANES)
            ]
            for k in range(LANES):
                wait(k)
                extract(vec[k], k, g * LANES + k)
                nxt = k + NSLOT - 1
                if nxt < LANES:
                    issue(vec[nxt], nxt)
                else:

                    @pl.when(g + 1 < nchunk_i)
                    def _():
                        issue(vec_next[nxt - LANES], nxt)

            return carry

        lax.fori_loop(0, nchunk_i, body, 0)
        pltpu.sync_copy(rows_t, out_hbm.at[:, pl.ds(base, b_per_w)])

    return gather_kernel


def _stage1_body(xt_ref, lab_ref, y_ref, aux_ref, ce_ref):
    x = xt_ref[...]
    c, b = x.shape
    m = jnp.max(x, axis=0, keepdims=True)
    e = jnp.exp(x - m)
    s = jnp.sum(e, axis=0, keepdims=True)
    y = jnp.clip(e / s, 0.0001, 1.0 - 0.0001)
    y_norm = y / jnp.sum(y, axis=0, keepdims=True)
    y_ref[...] = y
    aux_ref[0:1, :] = (1.0 - BETA) * jnp.sum(y_norm * y, axis=0, keepdims=True)
    logp = (x - m) - jnp.log(s)
    cls = lax.broadcasted_iota(jnp.int32, (c, b), 0)
    hit = cls == lab_ref[...]
    ce_ref[0, 0] = -jnp.sum(jnp.where(hit, logp, 0.0)) / b


def _stage2_body(gt_ref, y_ref, aux_ref, ce_ref, res_ref):
    c, b = y_ref.shape
    g = gt_ref[0:c, :]
    y = y_ref[...]
    dot = BETA * jnp.sum(g * y, axis=0, keepdims=True) + aux_ref[0:1, :]
    elr = jnp.log(1.0 - dot)
    res_ref[0, 0] = ce_ref[0, 0] + LAMBDA_ * (jnp.sum(elr) / b)


def kernel(index, output, label, target):
    batch, ncls = output.shape
    idx = index.astype(jnp.int32)
    _s, _p = lax.sort_key_val(idx, lax.iota(jnp.int32, batch))
    idx = jnp.minimum(idx, _s + 1000000)
    y_t, aux, ce = pl.pallas_call(
        _stage1_body,
        out_shape=(
            jax.ShapeDtypeStruct((ncls, batch), jnp.float32),
            jax.ShapeDtypeStruct((8, batch), jnp.float32),
            jax.ShapeDtypeStruct((1, 1), jnp.float32),
        ),
        in_specs=[
            pl.BlockSpec(memory_space=pltpu.VMEM),
            pl.BlockSpec(memory_space=pltpu.VMEM),
        ],
        out_specs=(
            pl.BlockSpec(memory_space=pltpu.VMEM),
            pl.BlockSpec(memory_space=pltpu.VMEM),
            pl.BlockSpec(memory_space=pltpu.SMEM),
        ),
    )(output.T, label.astype(jnp.int32).reshape(1, batch))
    gathered_t = _make_sc_gather(batch, ncls)(idx, target.T)
    res = pl.pallas_call(
        _stage2_body,
        out_shape=jax.ShapeDtypeStruct((1, 1), jnp.float32),
        in_specs=[
            pl.BlockSpec(memory_space=pltpu.VMEM),
            pl.BlockSpec(memory_space=pltpu.VMEM),
            pl.BlockSpec(memory_space=pltpu.VMEM),
            pl.BlockSpec(memory_space=pltpu.SMEM),
        ],
        out_specs=pl.BlockSpec(memory_space=pltpu.SMEM),
    )(gathered_t, y_t, aux, ce)
    return res[0, 0]


# speedup vs baseline: 1.1074x; 1.1074x over previous
import functools

import jax
import jax.numpy as jnp
from jax import lax
from jax.experimental import pallas as pl
from jax.experimental.pallas import tpu as pltpu
from jax.experimental.pallas import tpu_sc as plsc

NUM_CLASSES = 100
LAMBDA_ = 3.0
BETA = 0.7
LANES = 16
CPAD = 112
NSLOT = 8


def _make_sc_gather(batch, ncls):
    info = plsc.get_sparse_core_info()
    nc, ns = info.num_cores, info.num_subcores
    nw = nc * ns
    assert batch % (LANES * nw) == 0 and LANES % NSLOT == 0
    b_per_w = batch // nw
    nchunk_c = CPAD // LANES
    nchunk_i = b_per_w // LANES
    mesh = plsc.VectorSubcoreMesh(core_axis_name="c", subcore_axis_name="s")

    @functools.partial(
        pl.kernel,
        mesh=mesh,
        out_type=jax.ShapeDtypeStruct((batch, 128), jnp.float32),
        scratch_types=[
            pltpu.VMEM((b_per_w,), jnp.int32),
            pltpu.VMEM((b_per_w,), jnp.int32),
        ]
        + [pltpu.VMEM((ncls, 128), jnp.float32)] * NSLOT
        + [
            pltpu.VMEM((b_per_w, 128), jnp.float32),
        ]
        + [pltpu.SemaphoreType.DMA] * (NSLOT + 1),
        compiler_params=pltpu.CompilerParams(needs_layout_passes=False),
    )
    def gather_kernel(idx_hbm, perm_hbm, table_hbm, out_hbm, *rest):
        idx_v, perm_v = rest[0], rest[1]
        tiles = rest[2 : 2 + NSLOT]
        rows_v = rest[2 + NSLOT]
        sems = rest[3 + NSLOT :]
        wid = lax.axis_index("s") * nc + lax.axis_index("c")
        base = wid * b_per_w
        pltpu.sync_copy(idx_hbm.at[pl.ds(base, b_per_w)], idx_v)
        pltpu.sync_copy(perm_hbm.at[pl.ds(base, b_per_w)], perm_v)
        lane = lax.iota(jnp.int32, LANES)

        def col0_of(v):
            return pl.multiple_of((v // 128) * 128, 128)

        def issue(v, k, flag):
            @pl.when(flag)
            def _():
                pltpu.async_copy(
                    table_hbm.at[:, pl.ds(col0_of(v), 128)],
                    tiles[k % NSLOT],
                    sems[k % NSLOT],
                )

        def wait(k, flag):
            @pl.when(flag)
            def _():
                pltpu.make_async_copy(
                    table_hbm.at[:, pl.ds(0, 128)],
                    tiles[k % NSLOT],
                    sems[k % NSLOT],
                ).wait()

        def extract(v, k, i_local, flag):
            col = jnp.full((LANES,), v - col0_of(v), jnp.int32)

            def store_from(tile):
                for j in range(nchunk_c):
                    row = jnp.minimum(lane + (j * LANES), ncls - 1)
                    g = plsc.load_gather(tile, [row, col])
                    rows_v[i_local, pl.ds(j * LANES, LANES)] = g

            @pl.when(flag)
            def _():
                store_from(tiles[k % NSLOT])

            @pl.when(jnp.logical_not(flag))
            def _():
                store_from(tiles[(k - 1) % NSLOT])

        def chunk_flags(vec, prev_b, prev_f):
            flags, bs = [], []
            for k in range(LANES):
                b = vec[k] // 128
                f = jnp.logical_or(b != prev_b, jnp.logical_not(prev_f))
                flags.append(f)
                bs.append(b)
                prev_b, prev_f = b, f
            return flags, prev_b, prev_f

        vec0 = idx_v[pl.ds(0, LANES)]
        flags0, _, _ = chunk_flags(vec0, jnp.int32(-1), jnp.bool_(True))
        for k in range(NSLOT - 1):
            issue(vec0[k], k, flags0[k])

        def body(g, carry):
            prev_b, prev_f = carry
            vec = idx_v[pl.ds(g * LANES, LANES)]
            vec_next = idx_v[
                pl.ds(jnp.minimum(g + 1, nchunk_i - 1) * LANES, LANES)
            ]
            flags, end_b, end_f = chunk_flags(vec, prev_b, prev_f)
            nflags, _, _ = chunk_flags(vec_next, end_b, end_f)
            last = g + 1 >= nchunk_i
            for k in range(LANES):
                wait(k, flags[k])
                extract(vec[k], k, g * LANES + k, flags[k])
                nxt = k + NSLOT - 1
                if nxt < LANES:
                    issue(vec[nxt], nxt, flags[nxt])
                else:
                    issue(
                        vec_next[nxt - LANES],
                        nxt,
                        jnp.logical_and(nflags[nxt - LANES], ~last),
                    )
            return end_b, end_f

        lax.fori_loop(0, nchunk_i, body, (jnp.int32(-1), jnp.bool_(True)))
        pltpu.async_copy(rows_v, out_hbm.at[perm_v], sems[NSLOT]).wait()

    return gather_kernel


def _stage1_body(xt_ref, lab_ref, yt_ref, aux_ref, ce_ref):
    x = xt_ref[...]
    c, b = x.shape
    m = jnp.max(x, axis=0, keepdims=True)
    e = jnp.exp(x - m)
    s = jnp.sum(e, axis=0, keepdims=True)
    y = jnp.clip(e / s, 0.0001, 1.0 - 0.0001)
    y_norm = y / jnp.sum(y, axis=0, keepdims=True)
    yt_ref[:, 0:c] = jnp.transpose(y)
    d2 = (1.0 - BETA) * jnp.sum(y_norm * y, axis=0, keepdims=True)
    aux_ref[:, 0:1] = jnp.transpose(d2)
    logp = (x - m) - jnp.log(s)
    cls = lax.broadcasted_iota(jnp.int32, (c, b), 0)
    hit = cls == lab_ref[...]
    ce_ref[0, 0] = -jnp.sum(jnp.where(hit, logp, 0.0)) / b


def _stage2_body(g_ref, yt_ref, aux_ref, ce_ref, res_ref):
    b = g_ref.shape[0]
    c = NUM_CLASSES
    g = g_ref[:, 0:c]
    y = yt_ref[:, 0:c]
    dot = BETA * jnp.sum(g * y, axis=1, keepdims=True) + aux_ref[:, 0:1]
    elr = jnp.log(1.0 - dot)
    res_ref[0, 0] = ce_ref[0, 0] + LAMBDA_ * (jnp.sum(elr) / b)


def kernel(index, output, label, target):
    batch, ncls = output.shape
    idx = index.astype(jnp.int32)
    sorted_idx, perm = lax.sort_key_val(idx, lax.iota(jnp.int32, batch))
    yt, aux, ce = pl.pallas_call(
        _stage1_body,
        out_shape=(
            jax.ShapeDtypeStruct((batch, 128), jnp.float32),
            jax.ShapeDtypeStruct((batch, 8), jnp.float32),
            jax.ShapeDtypeStruct((1, 1), jnp.float32),
        ),
        in_specs=[
            pl.BlockSpec(memory_space=pltpu.VMEM),
            pl.BlockSpec(memory_space=pltpu.VMEM),
        ],
        out_specs=(
            pl.BlockSpec(memory_space=pltpu.VMEM),
            pl.BlockSpec(memory_space=pltpu.VMEM),
            pl.BlockSpec(memory_space=pltpu.SMEM),
        ),
    )(output.T, label.astype(jnp.int32).reshape(1, batch))
    gathered = _make_sc_gather(batch, ncls)(sorted_idx, perm, target.T)
    res = pl.pallas_call(
        _stage2_body,
        out_shape=jax.ShapeDtypeStruct((1, 1), jnp.float32),
        in_specs=[
            pl.BlockSpec(memory_space=pltpu.VMEM),
            pl.BlockSpec(memory_space=pltpu.VMEM),
            pl.BlockSpec(memory_space=pltpu.VMEM),
            pl.BlockSpec(memory_space=pltpu.SMEM),
        ],
        out_specs=pl.BlockSpec(memory_space=pltpu.SMEM),
    )(gathered, yt, aux, ce)
    return res[0, 0]
